# baseline (speedup 1.0000x reference)
import functools

import jax
import jax.numpy as jnp
from jax import lax
from jax.experimental import pallas as pl
from jax.experimental.pallas import tpu as pltpu
from jax.experimental.pallas import tpu_sc as plsc

FACTOR = 32
CHUNK = 128


def _gather_sc(batch, factor):
    info = plsc.get_sparse_core_info()
    nc, ns = info.num_cores, info.num_subcores
    nw = nc * ns
    b_per_w = batch // nw
    nchunk = b_per_w // CHUNK
    mesh = plsc.VectorSubcoreMesh(core_axis_name="c", subcore_axis_name="s")

    @functools.partial(
        pl.kernel,
        mesh=mesh,
        out_type=[
            jax.ShapeDtypeStruct((batch, factor), jnp.float32),
            jax.ShapeDtypeStruct((batch, factor), jnp.float32),
        ],
        scratch_types=[
            pltpu.VMEM((nchunk, CHUNK), jnp.int32),
            pltpu.VMEM((nchunk, CHUNK), jnp.int32),
            pltpu.VMEM((b_per_w, factor), jnp.float32),
            pltpu.VMEM((b_per_w, factor), jnp.float32),
            pltpu.SemaphoreType.DMA,
        ],
        compiler_params=pltpu.CompilerParams(use_tc_tiling_on_sc=False),
    )
    def gather_kernel(eu_hbm, ei_hbm, uidx_hbm, iidx_hbm, u_out, i_out,
                      uidx_v, iidx_v, urows_v, irows_v, sem):
        wid = lax.axis_index("s") * nc + lax.axis_index("c")
        base = wid * b_per_w
        pltpu.sync_copy(uidx_hbm.at[wid], uidx_v)
        pltpu.sync_copy(iidx_hbm.at[wid], iidx_v)
        copies = []
        for j in range(nchunk):
            copies.append(pltpu.async_copy(
                eu_hbm.at[uidx_v.at[j]],
                urows_v.at[pl.ds(j * CHUNK, CHUNK)], sem))
            copies.append(pltpu.async_copy(
                ei_hbm.at[iidx_v.at[j]],
                irows_v.at[pl.ds(j * CHUNK, CHUNK)], sem))
        for cp in copies:
            cp.wait()
        pltpu.sync_copy(urows_v, u_out.at[pl.ds(base, b_per_w)])
        pltpu.sync_copy(irows_v, i_out.at[pl.ds(base, b_per_w)])

    return gather_kernel, nw, nchunk


def _mlp_body(u_ref, i_ref, w0_ref, b0_ref, w1_ref, b1_ref, wo_ref, bo_ref,
              out_ref):
    u = u_ref[...]
    i = i_ref[...]
    w0 = w0_ref[...]
    x = jnp.dot(u, w0[:FACTOR, :], preferred_element_type=jnp.float32)
    x += jnp.dot(i, w0[FACTOR:, :], preferred_element_type=jnp.float32)
    x = jnp.maximum(x + b0_ref[...], 0.0)
    x = jnp.dot(x, w1_ref[...], preferred_element_type=jnp.float32)
    x = jnp.maximum(x + b1_ref[...], 0.0)
    pred = jnp.sum(x * wo_ref[...], axis=1) + bo_ref[0, 0]
    out_ref[...] = pred


@jax.jit
def kernel(user, item, embed_user, embed_item, W0, b0, W1, b1, Wo, bo):
    batch = user.shape[0]
    factor = embed_user.shape[1]
    gather_kernel, nw, nchunk = _gather_sc(batch, factor)

    uidx = user.astype(jnp.int32).reshape(nw, nchunk, CHUNK)
    iidx = item.astype(jnp.int32).reshape(nw, nchunk, CHUNK)
    u_rows, i_rows = gather_kernel(embed_user, embed_item, uidx, iidx)

    blk = 2048
    grid = (batch // blk,)
    out = pl.pallas_call(
        _mlp_body,
        grid=grid,
        in_specs=[
            pl.BlockSpec((blk, factor), lambda i: (i, 0)),
            pl.BlockSpec((blk, factor), lambda i: (i, 0)),
            pl.BlockSpec(W0.shape, lambda i: (0, 0)),
            pl.BlockSpec((1, W0.shape[1]), lambda i: (0, 0)),
            pl.BlockSpec(W1.shape, lambda i: (0, 0)),
            pl.BlockSpec((1, W1.shape[1]), lambda i: (0, 0)),
            pl.BlockSpec((1, Wo.shape[0]), lambda i: (0, 0)),
            pl.BlockSpec((1, 1), lambda i: (0, 0)),
        ],
        out_specs=pl.BlockSpec((blk,), lambda i: (i,)),
        out_shape=jax.ShapeDtypeStruct((batch,), jnp.float32),
    )(u_rows, i_rows, W0, b0.reshape(1, -1), W1, b1.reshape(1, -1),
      Wo.reshape(1, -1), bo.reshape(1, 1))
    return out
